# Initial kernel scaffold; baseline (speedup 1.0000x reference)
import jax
import jax.numpy as jnp
from jax import lax
from jax.experimental import pallas as pl


_ROWS_PER_BLOCK = 16


def _wta_block(in_ref, out_ref):
    x = in_ref[...]
    r, n = x.shape
    col = lax.broadcasted_iota(jnp.int32, (r, n), 1)
    m = jnp.max(x, axis=1, keepdims=True)
    idx = jnp.min(jnp.where(x == m, col, n), axis=1, keepdims=True)
    out_ref[...] = (col == idx).astype(x.dtype)


def kernel(tensor):
    b, n = tensor.shape
    grid = (b // _ROWS_PER_BLOCK,)
    return pl.pallas_call(
        _wta_block,
        grid=grid,
        in_specs=[pl.BlockSpec((_ROWS_PER_BLOCK, n), lambda i: (i, 0))],
        out_specs=pl.BlockSpec((_ROWS_PER_BLOCK, n), lambda i: (i, 0)),
        out_shape=jax.ShapeDtypeStruct((b, n), tensor.dtype),
    )(tensor)


# speedup vs baseline: 1.1156x; 1.1156x over previous
import functools

import jax
import jax.numpy as jnp
from jax import lax
from jax.experimental import pallas as pl
from jax.experimental.pallas import tpu as pltpu
from jax.experimental.pallas import tpu_sc as plsc

_B = 128
_N = 32768
_L = 16
_NC = 2
_NS = 16
_NW = _NC * _NS
_RPW = _B // _NW
_CHUNKS = _N // _L


def _wta_body(in_hbm, out_hbm, inbuf, zbuf, patch, sem_a, sem_b, sem_z, sem_p):
    wid = lax.axis_index("s") * _NC + lax.axis_index("c")
    base_row = wid * _RPW

    in_sems = (sem_a, sem_b)
    in_handles = [
        pltpu.async_copy(in_hbm.at[base_row + r], inbuf.at[r % 2], in_sems[r % 2])
        for r in range(2)
    ]

    zero16 = jnp.zeros((_L,), jnp.float32)

    def zero_body(i, _):
        zbuf[pl.ds(i * _L, _L)] = zero16
        return 0

    lax.fori_loop(0, _CHUNKS, zero_body, 0, unroll=8)
    z_handles = [
        pltpu.async_copy(zbuf, out_hbm.at[base_row + r], sem_z) for r in range(_RPW)
    ]

    lane = lax.iota(jnp.int32, _L)
    neg_inf = jnp.full((_L,), -jnp.inf, jnp.float32)

    def make_argmax_body(buf_i):
        def argmax_body(i, carry):
            runmax, runidx, curidx = carry
            x = inbuf[buf_i, pl.ds(i * _L, _L)]
            better = x > runmax
            runmax = jnp.where(better, x, runmax)
            runidx = jnp.where(better, curidx, runidx)
            return runmax, runidx, curidx + _L

        return argmax_body

    blks = []
    for r in range(_RPW):
        in_handles[r].wait()
        runmax, runidx, _ = lax.fori_loop(
            0, _CHUNKS, make_argmax_body(r % 2), (neg_inf, lane, lane), unroll=8
        )
        if r + 2 < _RPW:
            in_handles.append(
                pltpu.async_copy(
                    in_hbm.at[base_row + r + 2], inbuf.at[r % 2], in_sems[r % 2]
                )
            )
        best_v, best_i = runmax[0], runidx[0]
        for l in range(1, _L):
            v, i = runmax[l], runidx[l]
            better = jnp.logical_or(
                v > best_v, jnp.logical_and(v == best_v, i < best_i)
            )
            best_v = jnp.where(better, v, best_v)
            best_i = jnp.where(better, i, best_i)
        idx = best_i
        off = jnp.bitwise_and(idx, _L - 1)
        blk = pl.multiple_of(jnp.bitwise_and(idx, -_L), _L)
        patch[r] = jnp.where(lane == off, 1.0, 0.0).astype(jnp.float32)
        blks.append(blk)

    for h in z_handles:
        h.wait()
    p_handles = [
        pltpu.async_copy(
            patch.at[r], out_hbm.at[base_row + r, pl.ds(blks[r], _L)], sem_p
        )
        for r in range(_RPW)
    ]
    for h in p_handles:
        h.wait()


def kernel(tensor):
    mesh = plsc.VectorSubcoreMesh(
        core_axis_name="c", subcore_axis_name="s", num_cores=_NC, num_subcores=_NS
    )
    f = pl.kernel(
        _wta_body,
        out_type=jax.ShapeDtypeStruct((_B, _N), jnp.float32),
        mesh=mesh,
        scratch_types=[
            pltpu.VMEM((2, _N), jnp.float32),
            pltpu.VMEM((_N,), jnp.float32),
            pltpu.VMEM((_RPW, _L), jnp.float32),
            pltpu.SemaphoreType.DMA,
            pltpu.SemaphoreType.DMA,
            pltpu.SemaphoreType.DMA,
            pltpu.SemaphoreType.DMA,
        ],
    )
    return f(tensor)


# speedup vs baseline: 1.1395x; 1.0214x over previous
import functools

import jax
import jax.numpy as jnp
from jax import lax
from jax.experimental import pallas as pl
from jax.experimental.pallas import tpu as pltpu
from jax.experimental.pallas import tpu_sc as plsc

_B = 128
_N = 32768
_L = 16
_NC = 2
_NS = 16
_NW = _NC * _NS
_RPW = _B // _NW
_CHUNKS = _N // _L


def _wta_body(in_hbm, out_hbm, inbuf, zbuf, patch, sem_a, sem_b, sem_z, sem_p):
    wid = lax.axis_index("s") * _NC + lax.axis_index("c")
    base_row = wid * _RPW

    in_sems = (sem_a, sem_b)
    in_handles = [
        pltpu.async_copy(in_hbm.at[base_row + r], inbuf.at[r % 2], in_sems[r % 2])
        for r in range(2)
    ]

    zero16 = jnp.zeros((_L,), jnp.float32)

    def zero_body(i, _):
        zbuf[pl.ds(i * _L, _L)] = zero16
        return 0

    lax.fori_loop(0, _CHUNKS, zero_body, 0, unroll=8)
    z_handles = [
        pltpu.async_copy(zbuf, out_hbm.at[base_row + r], sem_z) for r in range(_RPW)
    ]

    lane = lax.iota(jnp.int32, _L)
    neg_inf = jnp.full((_L,), -jnp.inf, jnp.float32)

    def make_argmax_body(buf_i):
        def argmax_body(i, carry):
            runmax, runidx, curidx = carry
            x = inbuf[buf_i, pl.ds(i * _L, _L)]
            better = x > runmax
            runmax = jnp.where(better, x, runmax)
            runidx = jnp.where(better, curidx, runidx)
            return runmax, runidx, curidx + _L

        return argmax_body

    blks = []
    for r in range(_RPW):
        in_handles[r].wait()
        runmax, runidx, _ = lax.fori_loop(
            0, _CHUNKS, make_argmax_body(r % 2), (neg_inf, lane, lane), unroll=8
        )
        if r + 2 < _RPW:
            in_handles.append(
                pltpu.async_copy(
                    in_hbm.at[base_row + r + 2], inbuf.at[r % 2], in_sems[r % 2]
                )
            )
        best_v, best_i = runmax, runidx
        for s in (8, 4, 2, 1):
            perm = jnp.bitwise_xor(lane, s)
            ov = best_v.at[perm].get(mode="promise_in_bounds")
            oi = best_i.at[perm].get(mode="promise_in_bounds")
            better = jnp.logical_or(
                ov > best_v, jnp.logical_and(ov == best_v, oi < best_i)
            )
            best_v = jnp.where(better, ov, best_v)
            best_i = jnp.where(better, oi, best_i)
        idx = best_i[0]
        off = jnp.bitwise_and(idx, _L - 1)
        blk = pl.multiple_of(jnp.bitwise_and(idx, -_L), _L)
        patch[r] = jnp.where(lane == off, 1.0, 0.0).astype(jnp.float32)
        blks.append(blk)

    for h in z_handles:
        h.wait()
    p_handles = [
        pltpu.async_copy(
            patch.at[r], out_hbm.at[base_row + r, pl.ds(blks[r], _L)], sem_p
        )
        for r in range(_RPW)
    ]
    for h in p_handles:
        h.wait()


def kernel(tensor):
    mesh = plsc.VectorSubcoreMesh(
        core_axis_name="c", subcore_axis_name="s", num_cores=_NC, num_subcores=_NS
    )
    f = pl.kernel(
        _wta_body,
        out_type=jax.ShapeDtypeStruct((_B, _N), jnp.float32),
        mesh=mesh,
        scratch_types=[
            pltpu.VMEM((2, _N), jnp.float32),
            pltpu.VMEM((_N,), jnp.float32),
            pltpu.VMEM((_RPW, _L), jnp.float32),
            pltpu.SemaphoreType.DMA,
            pltpu.SemaphoreType.DMA,
            pltpu.SemaphoreType.DMA,
            pltpu.SemaphoreType.DMA,
        ],
    )
    return f(tensor)
